# baseline (speedup 1.0000x reference)
import functools

import jax
import jax.numpy as jnp
from jax import lax
from jax.experimental import pallas as pl
from jax.experimental.pallas import tpu as pltpu
from jax.experimental.pallas import tpu_sc as plsc

NUM_CORES = 2
NUM_SUBCORES = 16
LANES = 16

D = 128
EDGE_CHUNK = 96
NSLOT = 4
OCT = 8


def _fill(ref, n, value):
    vec = jnp.full((LANES,), value, jnp.float32)

    def body(i, _):
        ref[pl.ds(i * LANES, LANES)] = vec
        return 0

    lax.fori_loop(0, n // LANES, body, 0)


def _agg_one_direction(table, src, dst, zeros2d, zeros1d, out, cnt_out, acc,
                       cnt_sp, idx_s, idx_d, rows, ones_v, gsems, ssems, csems,
                       E, with_counts):
    sid = lax.axis_index("s")

    @pl.when(sid < 15)
    def _():
        r0 = pl.multiple_of(sid * 624, 8)
        pltpu.sync_copy(zeros2d.at[pl.ds(r0, 624)], acc.at[pl.ds(r0, 624)])

    @pl.when(sid == 15)
    def _():
        pltpu.sync_copy(zeros2d.at[pl.ds(9360, 640)], acc.at[pl.ds(9360, 640)])

    if with_counts:
        _fill(ones_v, EDGE_CHUNK, 1.0)

        @pl.when(sid == 0)
        def _():
            pltpu.sync_copy(zeros1d, cnt_sp)

    plsc.subcore_barrier()

    n_chunks = E // EDGE_CHUNK
    n_quads = n_chunks // NSLOT
    n_left = n_chunks - n_quads * NSLOT
    my_quads = (n_quads - sid + NUM_SUBCORES - 1) // NUM_SUBCORES

    def wait_scatter(b):
        pltpu.make_async_copy(rows[b], acc.at[idx_d[b]], ssems[b]).wait()
        if with_counts:
            pltpu.make_async_copy(ones_v.at[pl.ds(0, EDGE_CHUNK)],
                                  cnt_sp.at[idx_d[b]], csems[b]).wait()

    def start_scatter(b):
        pltpu.async_copy(rows[b], acc.at[idx_d[b]], ssems[b], add=True)
        if with_counts:
            pltpu.async_copy(ones_v.at[pl.ds(0, EDGE_CHUNK)],
                             cnt_sp.at[idx_d[b]], csems[b], add=True)

    def quad_body(k, _):
        q = sid + k * NUM_SUBCORES
        for b in range(NSLOT):
            @pl.when(k > 0)
            def _():
                wait_scatter(b)
            base = pl.multiple_of((q * NSLOT + b) * EDGE_CHUNK, 8)
            pltpu.sync_copy(src.at[pl.ds(base, EDGE_CHUNK)], idx_s[b])
            pltpu.sync_copy(dst.at[pl.ds(base, EDGE_CHUNK)], idx_d[b])
            pltpu.async_copy(table.at[idx_s[b]], rows[b], gsems[b])
        for b in range(NSLOT):
            pltpu.make_async_copy(table.at[idx_s[b]], rows[b],
                                  gsems[b]).wait()
            start_scatter(b)
        return 0

    lax.fori_loop(0, my_quads, quad_body, 0)

    for b in range(NSLOT):
        wait_scatter(b)

    for t in range(n_left):
        @pl.when(sid == 15)
        def _():
            base = pl.multiple_of((n_quads * NSLOT + t) * EDGE_CHUNK, 8)
            pltpu.sync_copy(src.at[pl.ds(base, EDGE_CHUNK)], idx_s[0])
            pltpu.sync_copy(dst.at[pl.ds(base, EDGE_CHUNK)], idx_d[0])
            pltpu.async_copy(table.at[idx_s[0]], rows[0], gsems[0]).wait()
            pltpu.sync_copy(rows[0], acc.at[idx_d[0]], add=True)
            if with_counts:
                pltpu.sync_copy(ones_v.at[pl.ds(0, EDGE_CHUNK)],
                                cnt_sp.at[idx_d[0]], add=True)

    plsc.subcore_barrier()

    @pl.when(sid < 15)
    def _():
        r0 = pl.multiple_of(sid * 624, 8)
        pltpu.sync_copy(acc.at[pl.ds(r0, 624)], out.at[pl.ds(r0, 624)])

    @pl.when(sid == 15)
    def _():
        pltpu.sync_copy(acc.at[pl.ds(9360, 640)], out.at[pl.ds(9360, 640)])

    if with_counts:
        @pl.when(sid == 0)
        def _():
            pltpu.sync_copy(cnt_sp, cnt_out)


def _make_agg(E, N, with_counts):
    assert E % EDGE_CHUNK == 0 and N == 10000
    mesh = plsc.VectorSubcoreMesh(core_axis_name="c", subcore_axis_name="s")
    out_type = [jax.ShapeDtypeStruct((N, D), jnp.float32),
                jax.ShapeDtypeStruct((N, D), jnp.float32)]
    if with_counts:
        out_type += [jax.ShapeDtypeStruct((N,), jnp.float32),
                     jax.ShapeDtypeStruct((N,), jnp.float32)]

    scratch = [
        pltpu.VMEM_SHARED((N, D), jnp.float32),
        pltpu.VMEM_SHARED((N,), jnp.float32),
        pltpu.VMEM((EDGE_CHUNK,), jnp.float32),
    ]
    scratch += [pltpu.VMEM((EDGE_CHUNK,), jnp.int32)] * (2 * NSLOT)
    scratch += [pltpu.VMEM((EDGE_CHUNK, D), jnp.float32)] * NSLOT
    scratch += [pltpu.SemaphoreType.DMA] * (3 * NSLOT)

    @functools.partial(
        pl.kernel,
        out_type=out_type,
        mesh=mesh,
        compiler_params=pltpu.CompilerParams(needs_layout_passes=False),
        scratch_types=scratch,
    )
    def agg(y_t, y_p, src_tp, dst_tp, src_pt, dst_pt, zeros2d, zeros1d, *rest):
        if with_counts:
            agg_p, agg_t, cnt_p, cnt_t = rest[:4]
            rest = rest[4:]
        else:
            agg_p, agg_t = rest[:2]
            cnt_p = cnt_t = None
            rest = rest[2:]
        acc, cnt_sp, ones_v = rest[:3]
        rest = rest[3:]
        idx_s = list(rest[:NSLOT])
        idx_d = list(rest[NSLOT:2 * NSLOT])
        rows = list(rest[2 * NSLOT:3 * NSLOT])
        gsems = list(rest[3 * NSLOT:4 * NSLOT])
        ssems = list(rest[4 * NSLOT:5 * NSLOT])
        csems = list(rest[5 * NSLOT:6 * NSLOT])
        cid = lax.axis_index("c")

        @pl.when(cid == 0)
        def _():
            _agg_one_direction(y_t, src_tp, dst_tp, zeros2d, zeros1d, agg_p,
                               cnt_p, acc, cnt_sp, idx_s, idx_d, rows, ones_v,
                               gsems, ssems, csems, E, with_counts)

        @pl.when(cid == 1)
        def _():
            _agg_one_direction(y_p, src_pt, dst_pt, zeros2d, zeros1d, agg_t,
                               cnt_t, acc, cnt_sp, idx_s, idx_d, rows, ones_v,
                               gsems, ssems, csems, E, with_counts)

    return agg


def _make_scoring(Q, N):
    mesh = plsc.VectorSubcoreMesh(core_axis_name="c", subcore_axis_name="s")
    C2 = 112
    n_full = Q // C2
    tail = Q - n_full * C2
    assert tail % LANES == 0 and (n_full * C2) % 8 == 0
    NW = NUM_CORES * NUM_SUBCORES
    assert n_full % 2 == 0
    n_pairs = n_full // 2

    @functools.partial(
        pl.kernel,
        out_type=jax.ShapeDtypeStruct((Q,), jnp.float32),
        mesh=mesh,
        compiler_params=pltpu.CompilerParams(needs_layout_passes=False),
        scratch_types=[
            pltpu.VMEM((C2,), jnp.int32),
            pltpu.VMEM((C2,), jnp.int32),
            pltpu.VMEM((C2,), jnp.int32),
            pltpu.VMEM((C2,), jnp.int32),
            pltpu.VMEM((C2, D), jnp.float32),
            pltpu.VMEM((C2, D), jnp.float32),
            pltpu.VMEM((C2, D), jnp.float32),
            pltpu.VMEM((C2, D), jnp.float32),
            pltpu.VMEM((C2,), jnp.float32),
            pltpu.SemaphoreType.DMA,
            pltpu.SemaphoreType.DMA,
            pltpu.SemaphoreType.DMA,
            pltpu.SemaphoreType.DMA,
        ],
    )
    def scoring(out_p, out_t, pidx, tidx, scores, pi0, pi1, ti0, ti1, pr0,
                pr1, tr0, tr1, sv, pa0, pa1, ta0, ta1):
        cid = lax.axis_index("c")
        sid = lax.axis_index("s")
        wid = sid * NUM_CORES + cid
        pi, ti = [pi0, pi1], [ti0, ti1]
        prows, trows = [pr0, pr1], [tr0, tr1]
        pa, ta = [pa0, pa1], [ta0, ta1]

        lane0 = lax.iota(jnp.int32, LANES) == 0

        def load_idx(b, c):
            base = pl.multiple_of(c * C2, 8)
            pltpu.sync_copy(pidx.at[pl.ds(base, C2)], pi[b])
            pltpu.sync_copy(tidx.at[pl.ds(base, C2)], ti[b])

        def start_gather(b):
            pltpu.async_copy(out_p.at[pi[b]], prows[b], pa[b])
            pltpu.async_copy(out_t.at[ti[b]], trows[b], ta[b])

        def compute(b, nq):
            def grp(g, _):
                row16 = lax.iota(jnp.int32, LANES) + g * LANES
                acc = jnp.zeros((LANES,), jnp.float32)
                for j in range(D):
                    colj = jnp.full((LANES,), j, jnp.int32)
                    pv = plsc.load_gather(prows[b], [row16, colj])
                    tv = plsc.load_gather(trows[b], [row16, colj])
                    acc = acc + pv * tv
                sv[pl.ds(g * LANES, LANES)] = acc
                return 0

            assert nq % LANES == 0
            lax.fori_loop(0, nq // LANES, grp, 0)

        my_pairs = (n_pairs - wid + NW - 1) // NW

        for b in (0, 1):
            load_idx(b, 2 * wid + b)
            start_gather(b)

        def pair_body(j, _):
            for b in (0, 1):
                c = 2 * (wid + j * NW) + b
                base = pl.multiple_of(c * C2, 8)
                pltpu.make_async_copy(out_p.at[pi[b]], prows[b], pa[b]).wait()
                pltpu.make_async_copy(out_t.at[ti[b]], trows[b], ta[b]).wait()
                compute(b, C2)
                pltpu.sync_copy(sv, scores.at[pl.ds(base, C2)])

                @pl.when(j + 1 < my_pairs)
                def _():
                    load_idx(b, 2 * (wid + (j + 1) * NW) + b)
                    start_gather(b)
            return 0

        lax.fori_loop(0, my_pairs, pair_body, 0)

        if tail:
            @pl.when(wid == NW - 1)
            def _():
                base = pl.multiple_of(n_full * C2, 8)
                pltpu.sync_copy(pidx.at[pl.ds(base, tail)],
                                pi[0].at[pl.ds(0, tail)])
                pltpu.sync_copy(tidx.at[pl.ds(base, tail)],
                                ti[0].at[pl.ds(0, tail)])
                start_gather(0)
                pltpu.make_async_copy(out_p.at[pi[0]], prows[0], pa[0]).wait()
                pltpu.make_async_copy(out_t.at[ti[0]], trows[0], ta[0]).wait()
                compute(0, tail)
                pltpu.sync_copy(sv.at[pl.ds(0, tail)],
                                scores.at[pl.ds(base, tail)])

    return scoring


_ROWS_BLK = 1000


def _transform_body(xp_ref, xt_ref, w_ref, op_ref, ot_ref):
    w = w_ref[...]
    op_ref[...] = jnp.dot(xp_ref[...], w, preferred_element_type=jnp.float32)
    ot_ref[...] = jnp.dot(xt_ref[...], w, preferred_element_type=jnp.float32)


def _tc_transform(xp, xt, wt):
    n = xp.shape[0]
    grid = (n // _ROWS_BLK,)
    blk = pl.BlockSpec((_ROWS_BLK, D), lambda i: (i, 0))
    blkw = pl.BlockSpec((D, D), lambda i: (0, 0))
    return pl.pallas_call(
        _transform_body,
        grid=grid,
        in_specs=[blk, blk, blkw],
        out_specs=[blk, blk],
        out_shape=[jax.ShapeDtypeStruct(xp.shape, jnp.float32),
                   jax.ShapeDtypeStruct(xt.shape, jnp.float32)],
    )(xp, xt, wt)


def _combine_one(agg, cnt, x, wr, wl, b):
    scale = 1.0 / jnp.maximum(cnt, 1.0)
    xn = jnp.maximum(
        agg * scale + jnp.dot(x, wr, preferred_element_type=jnp.float32) + b,
        0.0)
    return xn, jnp.dot(xn, wl, preferred_element_type=jnp.float32)


def _combine_body(ap_ref, at_ref, cp_ref, ct_ref, xp_ref, xt_ref, wrt_ref,
                  wlt_ref, b_ref, xnp_ref, xnt_ref, ynp_ref, ynt_ref):
    wr, wl, b = wrt_ref[...], wlt_ref[...], b_ref[...]
    xnp_ref[...], ynp_ref[...] = _combine_one(ap_ref[...], cp_ref[...],
                                              xp_ref[...], wr, wl, b)
    xnt_ref[...], ynt_ref[...] = _combine_one(at_ref[...], ct_ref[...],
                                              xt_ref[...], wr, wl, b)


def _tc_combine(ap, at, cp, ct, xp, xt, wrt, wlt_next, b):
    n = xp.shape[0]
    grid = (n // _ROWS_BLK,)
    blk = pl.BlockSpec((_ROWS_BLK, D), lambda i: (i, 0))
    blkc = pl.BlockSpec((_ROWS_BLK, 1), lambda i: (i, 0))
    blkw = pl.BlockSpec((D, D), lambda i: (0, 0))
    blkb = pl.BlockSpec((1, D), lambda i: (0, 0))
    sh = jax.ShapeDtypeStruct(xp.shape, jnp.float32)
    return pl.pallas_call(
        _combine_body,
        grid=grid,
        in_specs=[blk, blk, blkc, blkc, blk, blk, blkw, blkw, blkb],
        out_specs=[blk, blk, blk, blk],
        out_shape=[sh, sh, sh, sh],
    )(ap, at, cp, ct, xp, xt, wrt, wlt_next, b)


def _final_one(agg, cnt, x1, x0, wr, b):
    scale = 1.0 / jnp.maximum(cnt, 1.0)
    x2 = jnp.maximum(
        agg * scale + jnp.dot(x1, wr, preferred_element_type=jnp.float32) + b,
        0.0)
    return (x0 + x1 + x2) * (1.0 / 3.0)


def _final_body(ap_ref, at_ref, cp_ref, ct_ref, x1p_ref, x1t_ref, x0p_ref,
                x0t_ref, wrt_ref, b_ref, op_ref, ot_ref):
    wr, b = wrt_ref[...], b_ref[...]
    op_ref[...] = _final_one(ap_ref[...], cp_ref[...], x1p_ref[...],
                             x0p_ref[...], wr, b)
    ot_ref[...] = _final_one(at_ref[...], ct_ref[...], x1t_ref[...],
                             x0t_ref[...], wr, b)


def _tc_final(ap, at, cp, ct, x1p, x1t, x0p, x0t, wrt, b):
    n = x1p.shape[0]
    grid = (n // _ROWS_BLK,)
    blk = pl.BlockSpec((_ROWS_BLK, D), lambda i: (i, 0))
    blkc = pl.BlockSpec((_ROWS_BLK, 1), lambda i: (i, 0))
    blkw = pl.BlockSpec((D, D), lambda i: (0, 0))
    blkb = pl.BlockSpec((1, D), lambda i: (0, 0))
    sh = jax.ShapeDtypeStruct(x1p.shape, jnp.float32)
    return pl.pallas_call(
        _final_body,
        grid=grid,
        in_specs=[blk, blk, blkc, blkc, blk, blk, blk, blk, blkw, blkb],
        out_specs=[blk, blk],
        out_shape=[sh, sh],
    )(ap, at, cp, ct, x1p, x1t, x0p, x0t, wrt, b)


def kernel(playlist_n_id, track_n_id, edge_index_pt, edge_index_tp,
           edge_label_index, emb_playlist, emb_track, Wl, Wr, bl, br):
    NP = emb_playlist.shape[0]
    NT = emb_track.shape[0]
    E = edge_index_pt.shape[1]
    Q = edge_label_index.shape[1]
    assert NP == NT

    src_tp, dst_tp = edge_index_tp[0], edge_index_tp[1]
    src_pt, dst_pt = edge_index_pt[0], edge_index_pt[1]
    pidx, tidx = edge_label_index[0], edge_label_index[1]

    agg0 = _make_agg(E, NP, with_counts=True)
    agg1 = _make_agg(E, NP, with_counts=False)
    scoring = _make_scoring(Q, NP)

    zeros2d = jnp.zeros((NP, D), jnp.float32)
    zeros1d = jnp.zeros((NP,), jnp.float32)
    b0 = (bl[0] + br[0]).reshape(1, D)
    b1 = (bl[1] + br[1]).reshape(1, D)

    yp0, yt0 = _tc_transform(emb_playlist, emb_track, Wl[0].T)
    agg_p0, agg_t0, cnt_p, cnt_t = agg0(yt0, yp0, src_tp, dst_tp,
                                        src_pt, dst_pt, zeros2d, zeros1d)
    cp = cnt_p.reshape(NP, 1)
    ct = cnt_t.reshape(NP, 1)
    xp1, xt1, yp1, yt1 = _tc_combine(agg_p0, agg_t0, cp, ct, emb_playlist,
                                     emb_track, Wr[0].T, Wl[1].T, b0)

    agg_p1, agg_t1 = agg1(yt1, yp1, src_tp, dst_tp, src_pt, dst_pt,
                          zeros2d, zeros1d)
    outp, outt = _tc_final(agg_p1, agg_t1, cp, ct, xp1, xt1, emb_playlist,
                           emb_track, Wr[1].T, b1)

    return scoring(outp, outt, pidx, tidx)


# speedup vs baseline: 1.4453x; 1.4453x over previous
import functools

import jax
import jax.numpy as jnp
from jax import lax
from jax.experimental import pallas as pl
from jax.experimental.pallas import tpu as pltpu
from jax.experimental.pallas import tpu_sc as plsc

NUM_CORES = 2
NUM_SUBCORES = 16
LANES = 16

D = 128
EDGE_CHUNK = 96
NSLOT = 4
OCT = 8


def _fill(ref, n, value):
    vec = jnp.full((LANES,), value, jnp.float32)

    def body(i, _):
        ref[pl.ds(i * LANES, LANES)] = vec
        return 0

    lax.fori_loop(0, n // LANES, body, 0)


def _agg_one_direction(table, src, dst, zeros2d, zeros1d, out, cnt_out, acc,
                       cnt_sp, idx_s, idx_d, rows, ones_v, gsems, ssems, csems,
                       E, with_counts):
    sid = lax.axis_index("s")

    @pl.when(sid < 15)
    def _():
        r0 = pl.multiple_of(sid * 624, 8)
        pltpu.sync_copy(zeros2d.at[pl.ds(r0, 624)], acc.at[pl.ds(r0, 624)])

    @pl.when(sid == 15)
    def _():
        pltpu.sync_copy(zeros2d.at[pl.ds(9360, 640)], acc.at[pl.ds(9360, 640)])

    if with_counts:
        _fill(ones_v, EDGE_CHUNK, 1.0)

        @pl.when(sid == 0)
        def _():
            pltpu.sync_copy(zeros1d, cnt_sp)

    plsc.subcore_barrier()

    n_chunks = E // EDGE_CHUNK
    n_quads = n_chunks // NSLOT
    n_left = n_chunks - n_quads * NSLOT
    my_quads = (n_quads - sid + NUM_SUBCORES - 1) // NUM_SUBCORES

    def wait_scatter(b):
        pltpu.make_async_copy(rows[b], acc.at[idx_d[b]], ssems[b]).wait()
        if with_counts:
            pltpu.make_async_copy(ones_v.at[pl.ds(0, EDGE_CHUNK)],
                                  cnt_sp.at[idx_d[b]], csems[b]).wait()

    def start_scatter(b):
        pltpu.async_copy(rows[b], acc.at[idx_d[b]], ssems[b], add=True)
        if with_counts:
            pltpu.async_copy(ones_v.at[pl.ds(0, EDGE_CHUNK)],
                             cnt_sp.at[idx_d[b]], csems[b], add=True)

    def quad_body(k, _):
        q = sid + k * NUM_SUBCORES
        for b in range(NSLOT):
            @pl.when(k > 0)
            def _():
                wait_scatter(b)
            base = pl.multiple_of((q * NSLOT + b) * EDGE_CHUNK, 8)
            pltpu.sync_copy(src.at[pl.ds(base, EDGE_CHUNK)], idx_s[b])
            pltpu.sync_copy(dst.at[pl.ds(base, EDGE_CHUNK)], idx_d[b])
            pltpu.async_copy(table.at[idx_s[b]], rows[b], gsems[b])
        for b in range(NSLOT):
            pltpu.make_async_copy(table.at[idx_s[b]], rows[b],
                                  gsems[b]).wait()
            start_scatter(b)
        return 0

    lax.fori_loop(0, my_quads, quad_body, 0)

    for b in range(NSLOT):
        wait_scatter(b)

    for t in range(n_left):
        @pl.when(sid == 15)
        def _():
            base = pl.multiple_of((n_quads * NSLOT + t) * EDGE_CHUNK, 8)
            pltpu.sync_copy(src.at[pl.ds(base, EDGE_CHUNK)], idx_s[0])
            pltpu.sync_copy(dst.at[pl.ds(base, EDGE_CHUNK)], idx_d[0])
            pltpu.async_copy(table.at[idx_s[0]], rows[0], gsems[0]).wait()
            pltpu.sync_copy(rows[0], acc.at[idx_d[0]], add=True)
            if with_counts:
                pltpu.sync_copy(ones_v.at[pl.ds(0, EDGE_CHUNK)],
                                cnt_sp.at[idx_d[0]], add=True)

    plsc.subcore_barrier()

    @pl.when(sid < 15)
    def _():
        r0 = pl.multiple_of(sid * 624, 8)
        pltpu.sync_copy(acc.at[pl.ds(r0, 624)], out.at[pl.ds(r0, 624)])

    @pl.when(sid == 15)
    def _():
        pltpu.sync_copy(acc.at[pl.ds(9360, 640)], out.at[pl.ds(9360, 640)])

    if with_counts:
        @pl.when(sid == 0)
        def _():
            pltpu.sync_copy(cnt_sp, cnt_out)


def _make_agg(E, N, with_counts):
    assert E % EDGE_CHUNK == 0 and N == 10000
    mesh = plsc.VectorSubcoreMesh(core_axis_name="c", subcore_axis_name="s")
    out_type = [jax.ShapeDtypeStruct((N, D), jnp.float32),
                jax.ShapeDtypeStruct((N, D), jnp.float32)]
    if with_counts:
        out_type += [jax.ShapeDtypeStruct((N,), jnp.float32),
                     jax.ShapeDtypeStruct((N,), jnp.float32)]

    scratch = [
        pltpu.VMEM_SHARED((N, D), jnp.float32),
        pltpu.VMEM_SHARED((N,), jnp.float32),
        pltpu.VMEM((EDGE_CHUNK,), jnp.float32),
    ]
    scratch += [pltpu.VMEM((EDGE_CHUNK,), jnp.int32)] * (2 * NSLOT)
    scratch += [pltpu.VMEM((EDGE_CHUNK, D), jnp.float32)] * NSLOT
    scratch += [pltpu.SemaphoreType.DMA] * (3 * NSLOT)

    @functools.partial(
        pl.kernel,
        out_type=out_type,
        mesh=mesh,
        compiler_params=pltpu.CompilerParams(needs_layout_passes=False),
        scratch_types=scratch,
    )
    def agg(y_t, y_p, src_tp, dst_tp, src_pt, dst_pt, zeros2d, zeros1d, *rest):
        if with_counts:
            agg_p, agg_t, cnt_p, cnt_t = rest[:4]
            rest = rest[4:]
        else:
            agg_p, agg_t = rest[:2]
            cnt_p = cnt_t = None
            rest = rest[2:]
        acc, cnt_sp, ones_v = rest[:3]
        rest = rest[3:]
        idx_s = list(rest[:NSLOT])
        idx_d = list(rest[NSLOT:2 * NSLOT])
        rows = list(rest[2 * NSLOT:3 * NSLOT])
        gsems = list(rest[3 * NSLOT:4 * NSLOT])
        ssems = list(rest[4 * NSLOT:5 * NSLOT])
        csems = list(rest[5 * NSLOT:6 * NSLOT])
        cid = lax.axis_index("c")

        @pl.when(cid == 0)
        def _():
            _agg_one_direction(y_t, src_tp, dst_tp, zeros2d, zeros1d, agg_p,
                               cnt_p, acc, cnt_sp, idx_s, idx_d, rows, ones_v,
                               gsems, ssems, csems, E, with_counts)

        @pl.when(cid == 1)
        def _():
            _agg_one_direction(y_p, src_pt, dst_pt, zeros2d, zeros1d, agg_t,
                               cnt_t, acc, cnt_sp, idx_s, idx_d, rows, ones_v,
                               gsems, ssems, csems, E, with_counts)

    return agg


def _make_scoring(Q, N):
    mesh = plsc.VectorSubcoreMesh(core_axis_name="c", subcore_axis_name="s")
    C2 = 112
    n_full = Q // C2
    tail = Q - n_full * C2
    assert tail % LANES == 0 and (n_full * C2) % 8 == 0
    NW = NUM_CORES * NUM_SUBCORES
    assert n_full % 2 == 0
    n_pairs = n_full // 2

    @functools.partial(
        pl.kernel,
        out_type=jax.ShapeDtypeStruct((Q,), jnp.float32),
        mesh=mesh,
        compiler_params=pltpu.CompilerParams(needs_layout_passes=False),
        scratch_types=[
            pltpu.VMEM((C2,), jnp.int32),
            pltpu.VMEM((C2,), jnp.int32),
            pltpu.VMEM((C2,), jnp.int32),
            pltpu.VMEM((C2,), jnp.int32),
            pltpu.VMEM((C2, D), jnp.float32),
            pltpu.VMEM((C2, D), jnp.float32),
            pltpu.VMEM((C2, D), jnp.float32),
            pltpu.VMEM((C2, D), jnp.float32),
            pltpu.VMEM((C2,), jnp.float32),
            pltpu.SemaphoreType.DMA,
            pltpu.SemaphoreType.DMA,
            pltpu.SemaphoreType.DMA,
            pltpu.SemaphoreType.DMA,
        ],
    )
    def scoring(out_p, out_t, pidx, tidx, scores, pi0, pi1, ti0, ti1, pr0,
                pr1, tr0, tr1, sv, pa0, pa1, ta0, ta1):
        cid = lax.axis_index("c")
        sid = lax.axis_index("s")
        wid = sid * NUM_CORES + cid
        pi, ti = [pi0, pi1], [ti0, ti1]
        prows, trows = [pr0, pr1], [tr0, tr1]
        pa, ta = [pa0, pa1], [ta0, ta1]

        lane0 = lax.iota(jnp.int32, LANES) == 0

        def load_idx(b, c):
            base = pl.multiple_of(c * C2, 8)
            pltpu.sync_copy(pidx.at[pl.ds(base, C2)], pi[b])
            pltpu.sync_copy(tidx.at[pl.ds(base, C2)], ti[b])

        def start_gather(b):
            pltpu.async_copy(out_p.at[pi[b]], prows[b], pa[b])
            pltpu.async_copy(out_t.at[ti[b]], trows[b], ta[b])

        def compute(b, nq):
            def qbody(q, _):
                acc = jnp.zeros((LANES,), jnp.float32)
                for j in range(D // LANES):
                    acc = acc + (prows[b][q, pl.ds(j * LANES, LANES)]
                                 * trows[b][q, pl.ds(j * LANES, LANES)])
                s = jnp.sum(acc)
                plsc.store_scatter(sv, [jnp.full((LANES,), q, jnp.int32)],
                                   jnp.full((LANES,), s, jnp.float32),
                                   mask=lane0)
                return 0

            lax.fori_loop(0, nq, qbody, 0)

        my_pairs = (n_pairs - wid + NW - 1) // NW

        for b in (0, 1):
            load_idx(b, 2 * wid + b)
            start_gather(b)

        def pair_body(j, _):
            for b in (0, 1):
                c = 2 * (wid + j * NW) + b
                base = pl.multiple_of(c * C2, 8)
                pltpu.make_async_copy(out_p.at[pi[b]], prows[b], pa[b]).wait()
                pltpu.make_async_copy(out_t.at[ti[b]], trows[b], ta[b]).wait()
                compute(b, C2)
                pltpu.sync_copy(sv, scores.at[pl.ds(base, C2)])

                @pl.when(j + 1 < my_pairs)
                def _():
                    load_idx(b, 2 * (wid + (j + 1) * NW) + b)
                    start_gather(b)
            return 0

        lax.fori_loop(0, my_pairs, pair_body, 0)

        if tail:
            @pl.when(wid == NW - 1)
            def _():
                base = pl.multiple_of(n_full * C2, 8)
                pltpu.sync_copy(pidx.at[pl.ds(base, tail)],
                                pi[0].at[pl.ds(0, tail)])
                pltpu.sync_copy(tidx.at[pl.ds(base, tail)],
                                ti[0].at[pl.ds(0, tail)])
                start_gather(0)
                pltpu.make_async_copy(out_p.at[pi[0]], prows[0], pa[0]).wait()
                pltpu.make_async_copy(out_t.at[ti[0]], trows[0], ta[0]).wait()
                compute(0, tail)
                pltpu.sync_copy(sv.at[pl.ds(0, tail)],
                                scores.at[pl.ds(base, tail)])

    return scoring


_ROWS_BLK = 1000


def _transform_body(xp_ref, xt_ref, w_ref, op_ref, ot_ref):
    w = w_ref[...]
    op_ref[...] = jnp.dot(xp_ref[...], w, preferred_element_type=jnp.float32)
    ot_ref[...] = jnp.dot(xt_ref[...], w, preferred_element_type=jnp.float32)


def _tc_transform(xp, xt, wt):
    n = xp.shape[0]
    grid = (n // _ROWS_BLK,)
    blk = pl.BlockSpec((_ROWS_BLK, D), lambda i: (i, 0))
    blkw = pl.BlockSpec((D, D), lambda i: (0, 0))
    return pl.pallas_call(
        _transform_body,
        grid=grid,
        in_specs=[blk, blk, blkw],
        out_specs=[blk, blk],
        out_shape=[jax.ShapeDtypeStruct(xp.shape, jnp.float32),
                   jax.ShapeDtypeStruct(xt.shape, jnp.float32)],
    )(xp, xt, wt)


def _combine_one(agg, cnt, x, wr, wl, b):
    scale = 1.0 / jnp.maximum(cnt, 1.0)
    xn = jnp.maximum(
        agg * scale + jnp.dot(x, wr, preferred_element_type=jnp.float32) + b,
        0.0)
    return xn, jnp.dot(xn, wl, preferred_element_type=jnp.float32)


def _combine_body(ap_ref, at_ref, cp_ref, ct_ref, xp_ref, xt_ref, wrt_ref,
                  wlt_ref, b_ref, xnp_ref, xnt_ref, ynp_ref, ynt_ref):
    wr, wl, b = wrt_ref[...], wlt_ref[...], b_ref[...]
    xnp_ref[...], ynp_ref[...] = _combine_one(ap_ref[...], cp_ref[...],
                                              xp_ref[...], wr, wl, b)
    xnt_ref[...], ynt_ref[...] = _combine_one(at_ref[...], ct_ref[...],
                                              xt_ref[...], wr, wl, b)


def _tc_combine(ap, at, cp, ct, xp, xt, wrt, wlt_next, b):
    n = xp.shape[0]
    grid = (n // _ROWS_BLK,)
    blk = pl.BlockSpec((_ROWS_BLK, D), lambda i: (i, 0))
    blkc = pl.BlockSpec((_ROWS_BLK, 1), lambda i: (i, 0))
    blkw = pl.BlockSpec((D, D), lambda i: (0, 0))
    blkb = pl.BlockSpec((1, D), lambda i: (0, 0))
    sh = jax.ShapeDtypeStruct(xp.shape, jnp.float32)
    return pl.pallas_call(
        _combine_body,
        grid=grid,
        in_specs=[blk, blk, blkc, blkc, blk, blk, blkw, blkw, blkb],
        out_specs=[blk, blk, blk, blk],
        out_shape=[sh, sh, sh, sh],
    )(ap, at, cp, ct, xp, xt, wrt, wlt_next, b)


def _final_one(agg, cnt, x1, x0, wr, b):
    scale = 1.0 / jnp.maximum(cnt, 1.0)
    x2 = jnp.maximum(
        agg * scale + jnp.dot(x1, wr, preferred_element_type=jnp.float32) + b,
        0.0)
    return (x0 + x1 + x2) * (1.0 / 3.0)


def _final_body(ap_ref, at_ref, cp_ref, ct_ref, x1p_ref, x1t_ref, x0p_ref,
                x0t_ref, wrt_ref, b_ref, op_ref, ot_ref):
    wr, b = wrt_ref[...], b_ref[...]
    op_ref[...] = _final_one(ap_ref[...], cp_ref[...], x1p_ref[...],
                             x0p_ref[...], wr, b)
    ot_ref[...] = _final_one(at_ref[...], ct_ref[...], x1t_ref[...],
                             x0t_ref[...], wr, b)


def _tc_final(ap, at, cp, ct, x1p, x1t, x0p, x0t, wrt, b):
    n = x1p.shape[0]
    grid = (n // _ROWS_BLK,)
    blk = pl.BlockSpec((_ROWS_BLK, D), lambda i: (i, 0))
    blkc = pl.BlockSpec((_ROWS_BLK, 1), lambda i: (i, 0))
    blkw = pl.BlockSpec((D, D), lambda i: (0, 0))
    blkb = pl.BlockSpec((1, D), lambda i: (0, 0))
    sh = jax.ShapeDtypeStruct(x1p.shape, jnp.float32)
    return pl.pallas_call(
        _final_body,
        grid=grid,
        in_specs=[blk, blk, blkc, blkc, blk, blk, blk, blk, blkw, blkb],
        out_specs=[blk, blk],
        out_shape=[sh, sh],
    )(ap, at, cp, ct, x1p, x1t, x0p, x0t, wrt, b)


def kernel(playlist_n_id, track_n_id, edge_index_pt, edge_index_tp,
           edge_label_index, emb_playlist, emb_track, Wl, Wr, bl, br):
    NP = emb_playlist.shape[0]
    NT = emb_track.shape[0]
    E = edge_index_pt.shape[1]
    Q = edge_label_index.shape[1]
    assert NP == NT

    src_tp, dst_tp = edge_index_tp[0], edge_index_tp[1]
    src_pt, dst_pt = edge_index_pt[0], edge_index_pt[1]
    pidx, tidx = edge_label_index[0], edge_label_index[1]

    agg0 = _make_agg(E, NP, with_counts=True)
    agg1 = _make_agg(E, NP, with_counts=False)
    scoring = _make_scoring(Q, NP)

    zeros2d = jnp.zeros((NP, D), jnp.float32)
    zeros1d = jnp.zeros((NP,), jnp.float32)
    b0 = (bl[0] + br[0]).reshape(1, D)
    b1 = (bl[1] + br[1]).reshape(1, D)

    yp0, yt0 = _tc_transform(emb_playlist, emb_track, Wl[0].T)
    agg_p0, agg_t0, cnt_p, cnt_t = agg0(yt0, yp0, src_tp, dst_tp,
                                        src_pt, dst_pt, zeros2d, zeros1d)
    cp = cnt_p.reshape(NP, 1)
    ct = cnt_t.reshape(NP, 1)
    xp1, xt1, yp1, yt1 = _tc_combine(agg_p0, agg_t0, cp, ct, emb_playlist,
                                     emb_track, Wr[0].T, Wl[1].T, b0)

    agg_p1, agg_t1 = agg1(yt1, yp1, src_tp, dst_tp, src_pt, dst_pt,
                          zeros2d, zeros1d)
    outp, outt = _tc_final(agg_p1, agg_t1, cp, ct, xp1, xt1, emb_playlist,
                           emb_track, Wr[1].T, b1)

    return scoring(outp, outt, pidx, tidx)


# speedup vs baseline: 1.4462x; 1.0006x over previous
import functools

import jax
import jax.numpy as jnp
from jax import lax
from jax.experimental import pallas as pl
from jax.experimental.pallas import tpu as pltpu
from jax.experimental.pallas import tpu_sc as plsc

NUM_CORES = 2
NUM_SUBCORES = 16
LANES = 16

D = 128
EDGE_CHUNK = 96
NSLOT = 4


def _fill(ref, n, value):
    vec = jnp.full((LANES,), value, jnp.float32)

    def body(i, _):
        ref[pl.ds(i * LANES, LANES)] = vec
        return 0

    lax.fori_loop(0, n // LANES, body, 0)


def _agg_one_direction(table, src, dst, zeros2d, zeros1d, out, cnt_out, acc,
                       cnt_sp, idx_s, idx_d, rows, ones_v, gsems, ssems, csems,
                       E, with_counts):
    sid = lax.axis_index("s")

    @pl.when(sid < 15)
    def _():
        r0 = pl.multiple_of(sid * 624, 8)
        pltpu.sync_copy(zeros2d.at[pl.ds(r0, 624)], acc.at[pl.ds(r0, 624)])

    @pl.when(sid == 15)
    def _():
        pltpu.sync_copy(zeros2d.at[pl.ds(9360, 640)], acc.at[pl.ds(9360, 640)])

    if with_counts:
        _fill(ones_v, EDGE_CHUNK, 1.0)

        @pl.when(sid == 0)
        def _():
            pltpu.sync_copy(zeros1d, cnt_sp)

    plsc.subcore_barrier()

    n_chunks = E // EDGE_CHUNK
    n_quads = n_chunks // NSLOT
    n_left = n_chunks - n_quads * NSLOT
    my_quads = (n_quads - sid + NUM_SUBCORES - 1) // NUM_SUBCORES

    def wait_scatter(b):
        pltpu.make_async_copy(rows[b], acc.at[idx_d[b]], ssems[b]).wait()
        if with_counts:
            pltpu.make_async_copy(ones_v.at[pl.ds(0, EDGE_CHUNK)],
                                  cnt_sp.at[idx_d[b]], csems[b]).wait()

    def start_scatter(b):
        pltpu.async_copy(rows[b], acc.at[idx_d[b]], ssems[b], add=True)
        if with_counts:
            pltpu.async_copy(ones_v.at[pl.ds(0, EDGE_CHUNK)],
                             cnt_sp.at[idx_d[b]], csems[b], add=True)

    def quad_body(k, _):
        q = sid + k * NUM_SUBCORES
        for b in range(NSLOT):
            @pl.when(k > 0)
            def _():
                wait_scatter(b)
            base = pl.multiple_of((q * NSLOT + b) * EDGE_CHUNK, 8)
            pltpu.sync_copy(src.at[pl.ds(base, EDGE_CHUNK)], idx_s[b])
            pltpu.sync_copy(dst.at[pl.ds(base, EDGE_CHUNK)], idx_d[b])
            pltpu.async_copy(table.at[idx_s[b]], rows[b], gsems[b])
        for b in range(NSLOT):
            pltpu.make_async_copy(table.at[idx_s[b]], rows[b],
                                  gsems[b]).wait()
            start_scatter(b)
        return 0

    lax.fori_loop(0, my_quads, quad_body, 0)

    for b in range(NSLOT):
        wait_scatter(b)

    for t in range(n_left):
        @pl.when(sid == 15)
        def _():
            base = pl.multiple_of((n_quads * NSLOT + t) * EDGE_CHUNK, 8)
            pltpu.sync_copy(src.at[pl.ds(base, EDGE_CHUNK)], idx_s[0])
            pltpu.sync_copy(dst.at[pl.ds(base, EDGE_CHUNK)], idx_d[0])
            pltpu.async_copy(table.at[idx_s[0]], rows[0], gsems[0]).wait()
            pltpu.sync_copy(rows[0], acc.at[idx_d[0]], add=True)
            if with_counts:
                pltpu.sync_copy(ones_v.at[pl.ds(0, EDGE_CHUNK)],
                                cnt_sp.at[idx_d[0]], add=True)

    plsc.subcore_barrier()

    @pl.when(sid < 15)
    def _():
        r0 = pl.multiple_of(sid * 624, 8)
        pltpu.sync_copy(acc.at[pl.ds(r0, 624)], out.at[pl.ds(r0, 624)])

    @pl.when(sid == 15)
    def _():
        pltpu.sync_copy(acc.at[pl.ds(9360, 640)], out.at[pl.ds(9360, 640)])

    if with_counts:
        @pl.when(sid == 0)
        def _():
            pltpu.sync_copy(cnt_sp, cnt_out)


def _make_agg(E, N, with_counts):
    assert E % EDGE_CHUNK == 0 and N == 10000
    mesh = plsc.VectorSubcoreMesh(core_axis_name="c", subcore_axis_name="s")
    out_type = [jax.ShapeDtypeStruct((N, D), jnp.float32),
                jax.ShapeDtypeStruct((N, D), jnp.float32)]
    if with_counts:
        out_type += [jax.ShapeDtypeStruct((N,), jnp.float32),
                     jax.ShapeDtypeStruct((N,), jnp.float32)]

    scratch = [
        pltpu.VMEM_SHARED((N, D), jnp.float32),
        pltpu.VMEM_SHARED((N,), jnp.float32),
        pltpu.VMEM((EDGE_CHUNK,), jnp.float32),
    ]
    scratch += [pltpu.VMEM((EDGE_CHUNK,), jnp.int32)] * (2 * NSLOT)
    scratch += [pltpu.VMEM((EDGE_CHUNK, D), jnp.float32)] * NSLOT
    scratch += [pltpu.SemaphoreType.DMA] * (3 * NSLOT)

    @functools.partial(
        pl.kernel,
        out_type=out_type,
        mesh=mesh,
        compiler_params=pltpu.CompilerParams(needs_layout_passes=False),
        scratch_types=scratch,
    )
    def agg(y_t, y_p, src_tp, dst_tp, src_pt, dst_pt, zeros2d, zeros1d, *rest):
        if with_counts:
            agg_p, agg_t, cnt_p, cnt_t = rest[:4]
            rest = rest[4:]
        else:
            agg_p, agg_t = rest[:2]
            cnt_p = cnt_t = None
            rest = rest[2:]
        acc, cnt_sp, ones_v = rest[:3]
        rest = rest[3:]
        idx_s = list(rest[:NSLOT])
        idx_d = list(rest[NSLOT:2 * NSLOT])
        rows = list(rest[2 * NSLOT:3 * NSLOT])
        gsems = list(rest[3 * NSLOT:4 * NSLOT])
        ssems = list(rest[4 * NSLOT:5 * NSLOT])
        csems = list(rest[5 * NSLOT:6 * NSLOT])
        cid = lax.axis_index("c")

        @pl.when(cid == 0)
        def _():
            _agg_one_direction(y_t, src_tp, dst_tp, zeros2d, zeros1d, agg_p,
                               cnt_p, acc, cnt_sp, idx_s, idx_d, rows, ones_v,
                               gsems, ssems, csems, E, with_counts)

        @pl.when(cid == 1)
        def _():
            _agg_one_direction(y_p, src_pt, dst_pt, zeros2d, zeros1d, agg_t,
                               cnt_t, acc, cnt_sp, idx_s, idx_d, rows, ones_v,
                               gsems, ssems, csems, E, with_counts)

    return agg


def _make_scoring(Q, N):
    mesh = plsc.VectorSubcoreMesh(core_axis_name="c", subcore_axis_name="s")
    C2 = 112
    n_full = Q // C2
    tail = Q - n_full * C2
    assert tail % LANES == 0 and (n_full * C2) % 8 == 0
    NW = NUM_CORES * NUM_SUBCORES
    assert n_full % 2 == 0
    n_pairs = n_full // 2

    @functools.partial(
        pl.kernel,
        out_type=jax.ShapeDtypeStruct((Q,), jnp.float32),
        mesh=mesh,
        compiler_params=pltpu.CompilerParams(needs_layout_passes=False),
        scratch_types=[
            pltpu.VMEM((C2,), jnp.int32),
            pltpu.VMEM((C2,), jnp.int32),
            pltpu.VMEM((C2,), jnp.int32),
            pltpu.VMEM((C2,), jnp.int32),
            pltpu.VMEM((C2, D), jnp.float32),
            pltpu.VMEM((C2, D), jnp.float32),
            pltpu.VMEM((C2, D), jnp.float32),
            pltpu.VMEM((C2, D), jnp.float32),
            pltpu.VMEM((C2,), jnp.float32),
            pltpu.SemaphoreType.DMA,
            pltpu.SemaphoreType.DMA,
            pltpu.SemaphoreType.DMA,
            pltpu.SemaphoreType.DMA,
        ],
    )
    def scoring(out_p, out_t, pidx, tidx, scores, pi0, pi1, ti0, ti1, pr0,
                pr1, tr0, tr1, sv, pa0, pa1, ta0, ta1):
        cid = lax.axis_index("c")
        sid = lax.axis_index("s")
        wid = sid * NUM_CORES + cid
        pi, ti = [pi0, pi1], [ti0, ti1]
        prows, trows = [pr0, pr1], [tr0, tr1]
        pa, ta = [pa0, pa1], [ta0, ta1]

        lane0 = lax.iota(jnp.int32, LANES) == 0

        def load_idx(b, c):
            base = pl.multiple_of(c * C2, 8)
            pltpu.sync_copy(pidx.at[pl.ds(base, C2)], pi[b])
            pltpu.sync_copy(tidx.at[pl.ds(base, C2)], ti[b])

        def start_gather(b):
            pltpu.async_copy(out_p.at[pi[b]], prows[b], pa[b])
            pltpu.async_copy(out_t.at[ti[b]], trows[b], ta[b])

        def compute(b, nq):
            def qbody(q, _):
                acc = jnp.zeros((LANES,), jnp.float32)
                for j in range(D // LANES):
                    acc = acc + (prows[b][q, pl.ds(j * LANES, LANES)]
                                 * trows[b][q, pl.ds(j * LANES, LANES)])
                s = jnp.sum(acc)
                plsc.store_scatter(sv, [jnp.full((LANES,), q, jnp.int32)],
                                   jnp.full((LANES,), s, jnp.float32),
                                   mask=lane0)
                return 0

            lax.fori_loop(0, nq, qbody, 0)

        my_pairs = (n_pairs - wid + NW - 1) // NW

        for b in (0, 1):
            load_idx(b, 2 * wid + b)
            start_gather(b)

        def pair_body(j, _):
            for b in (0, 1):
                c = 2 * (wid + j * NW) + b
                base = pl.multiple_of(c * C2, 8)
                pltpu.make_async_copy(out_p.at[pi[b]], prows[b], pa[b]).wait()
                pltpu.make_async_copy(out_t.at[ti[b]], trows[b], ta[b]).wait()
                compute(b, C2)
                pltpu.sync_copy(sv, scores.at[pl.ds(base, C2)])

                @pl.when(j + 1 < my_pairs)
                def _():
                    load_idx(b, 2 * (wid + (j + 1) * NW) + b)
                    start_gather(b)
            return 0

        lax.fori_loop(0, my_pairs, pair_body, 0)

        if tail:
            @pl.when(wid == NW - 1)
            def _():
                base = pl.multiple_of(n_full * C2, 8)
                pltpu.sync_copy(pidx.at[pl.ds(base, tail)],
                                pi[0].at[pl.ds(0, tail)])
                pltpu.sync_copy(tidx.at[pl.ds(base, tail)],
                                ti[0].at[pl.ds(0, tail)])
                start_gather(0)
                pltpu.make_async_copy(out_p.at[pi[0]], prows[0], pa[0]).wait()
                pltpu.make_async_copy(out_t.at[ti[0]], trows[0], ta[0]).wait()
                compute(0, tail)
                pltpu.sync_copy(sv.at[pl.ds(0, tail)],
                                scores.at[pl.ds(base, tail)])

    return scoring


_ROWS_BLK = 1000


def _transform_body(xp_ref, xt_ref, w_ref, op_ref, ot_ref):
    w = w_ref[...]
    op_ref[...] = jnp.dot(xp_ref[...], w, preferred_element_type=jnp.float32)
    ot_ref[...] = jnp.dot(xt_ref[...], w, preferred_element_type=jnp.float32)


def _tc_transform(xp, xt, wt):
    n = xp.shape[0]
    grid = (n // _ROWS_BLK,)
    blk = pl.BlockSpec((_ROWS_BLK, D), lambda i: (i, 0))
    blkw = pl.BlockSpec((D, D), lambda i: (0, 0))
    return pl.pallas_call(
        _transform_body,
        grid=grid,
        in_specs=[blk, blk, blkw],
        out_specs=[blk, blk],
        out_shape=[jax.ShapeDtypeStruct(xp.shape, jnp.float32),
                   jax.ShapeDtypeStruct(xt.shape, jnp.float32)],
    )(xp, xt, wt)


def _combine_one(agg, cnt, x, wr, wl, b):
    scale = 1.0 / jnp.maximum(cnt, 1.0)
    xn = jnp.maximum(
        agg * scale + jnp.dot(x, wr, preferred_element_type=jnp.float32) + b,
        0.0)
    return xn, jnp.dot(xn, wl, preferred_element_type=jnp.float32)


def _combine_body(ap_ref, at_ref, cp_ref, ct_ref, xp_ref, xt_ref, wrt_ref,
                  wlt_ref, b_ref, xnp_ref, xnt_ref, ynp_ref, ynt_ref):
    wr, wl, b = wrt_ref[...], wlt_ref[...], b_ref[...]
    xnp_ref[...], ynp_ref[...] = _combine_one(ap_ref[...], cp_ref[...],
                                              xp_ref[...], wr, wl, b)
    xnt_ref[...], ynt_ref[...] = _combine_one(at_ref[...], ct_ref[...],
                                              xt_ref[...], wr, wl, b)


def _tc_combine(ap, at, cp, ct, xp, xt, wrt, wlt_next, b):
    n = xp.shape[0]
    grid = (n // _ROWS_BLK,)
    blk = pl.BlockSpec((_ROWS_BLK, D), lambda i: (i, 0))
    blkc = pl.BlockSpec((_ROWS_BLK, 1), lambda i: (i, 0))
    blkw = pl.BlockSpec((D, D), lambda i: (0, 0))
    blkb = pl.BlockSpec((1, D), lambda i: (0, 0))
    sh = jax.ShapeDtypeStruct(xp.shape, jnp.float32)
    return pl.pallas_call(
        _combine_body,
        grid=grid,
        in_specs=[blk, blk, blkc, blkc, blk, blk, blkw, blkw, blkb],
        out_specs=[blk, blk, blk, blk],
        out_shape=[sh, sh, sh, sh],
    )(ap, at, cp, ct, xp, xt, wrt, wlt_next, b)


def _final_one(agg, cnt, x1, x0, wr, b):
    scale = 1.0 / jnp.maximum(cnt, 1.0)
    x2 = jnp.maximum(
        agg * scale + jnp.dot(x1, wr, preferred_element_type=jnp.float32) + b,
        0.0)
    return (x0 + x1 + x2) * (1.0 / 3.0)


def _final_body(ap_ref, at_ref, cp_ref, ct_ref, x1p_ref, x1t_ref, x0p_ref,
                x0t_ref, wrt_ref, b_ref, op_ref, ot_ref):
    wr, b = wrt_ref[...], b_ref[...]
    op_ref[...] = _final_one(ap_ref[...], cp_ref[...], x1p_ref[...],
                             x0p_ref[...], wr, b)
    ot_ref[...] = _final_one(at_ref[...], ct_ref[...], x1t_ref[...],
                             x0t_ref[...], wr, b)


def _tc_final(ap, at, cp, ct, x1p, x1t, x0p, x0t, wrt, b):
    n = x1p.shape[0]
    grid = (n // _ROWS_BLK,)
    blk = pl.BlockSpec((_ROWS_BLK, D), lambda i: (i, 0))
    blkc = pl.BlockSpec((_ROWS_BLK, 1), lambda i: (i, 0))
    blkw = pl.BlockSpec((D, D), lambda i: (0, 0))
    blkb = pl.BlockSpec((1, D), lambda i: (0, 0))
    sh = jax.ShapeDtypeStruct(x1p.shape, jnp.float32)
    return pl.pallas_call(
        _final_body,
        grid=grid,
        in_specs=[blk, blk, blkc, blkc, blk, blk, blk, blk, blkw, blkb],
        out_specs=[blk, blk],
        out_shape=[sh, sh],
    )(ap, at, cp, ct, x1p, x1t, x0p, x0t, wrt, b)


def kernel(playlist_n_id, track_n_id, edge_index_pt, edge_index_tp,
           edge_label_index, emb_playlist, emb_track, Wl, Wr, bl, br):
    NP = emb_playlist.shape[0]
    NT = emb_track.shape[0]
    E = edge_index_pt.shape[1]
    Q = edge_label_index.shape[1]
    assert NP == NT

    src_tp, dst_tp = edge_index_tp[0], edge_index_tp[1]
    src_pt, dst_pt = edge_index_pt[0], edge_index_pt[1]
    pidx, tidx = edge_label_index[0], edge_label_index[1]

    agg0 = _make_agg(E, NP, with_counts=True)
    agg1 = _make_agg(E, NP, with_counts=False)
    scoring = _make_scoring(Q, NP)

    zeros2d = jnp.zeros((NP, D), jnp.float32)
    zeros1d = jnp.zeros((NP,), jnp.float32)
    b0 = (bl[0] + br[0]).reshape(1, D)
    b1 = (bl[1] + br[1]).reshape(1, D)

    yp0, yt0 = _tc_transform(emb_playlist, emb_track, Wl[0].T)
    agg_p0, agg_t0, cnt_p, cnt_t = agg0(yt0, yp0, src_tp, dst_tp,
                                        src_pt, dst_pt, zeros2d, zeros1d)
    cp = cnt_p.reshape(NP, 1)
    ct = cnt_t.reshape(NP, 1)
    xp1, xt1, yp1, yt1 = _tc_combine(agg_p0, agg_t0, cp, ct, emb_playlist,
                                     emb_track, Wr[0].T, Wl[1].T, b0)

    agg_p1, agg_t1 = agg1(yt1, yp1, src_tp, dst_tp, src_pt, dst_pt,
                          zeros2d, zeros1d)
    outp, outt = _tc_final(agg_p1, agg_t1, cp, ct, xp1, xt1, emb_playlist,
                           emb_track, Wr[1].T, b1)

    return scoring(outp, outt, pidx, tidx)
